# baseline (speedup 1.0000x reference)
import jax
import jax.numpy as jnp
from jax.experimental import pallas as pl
from jax.experimental.pallas import tpu as pltpu


_H = 256
_B = 32
_T = 128
_CLEN = 16
_CE = 64
_HC = 128
_TC = 8
_NEG = 1e10


def _encode_proj_kernel(ids_ref, wx_ref, tm_ref, ce_ref,
                        wc_ref, bc_ref, ww_ref, wch_ref, mpos_ref, b_ref,
                        gx_ref):
    ids = ids_ref[...]
    cid = ids[:, :_CLEN]
    rt = cid.shape[0]

    iota_c = jax.lax.broadcasted_iota(jnp.int32, (1, 1, 128), 2)
    oh_c = (cid[:, :, None] == iota_c).astype(jnp.bfloat16)
    x = jax.lax.dot_general(oh_c, ce_ref[...],
                            dimension_numbers=(((2,), (0,)), ((), ())),
                            preferred_element_type=jnp.float32)
    x = x.astype(jnp.bfloat16)

    z = jnp.zeros((rt, 1, _CE), x.dtype)
    patches = jnp.concatenate(
        [jnp.concatenate([z, x[:, :-1, :]], axis=1),
         x,
         jnp.concatenate([x[:, 1:, :], z], axis=1)], axis=2)
    h = jax.lax.dot_general(patches, wc_ref[...],
                            dimension_numbers=(((2,), (0,)), ((), ())),
                            preferred_element_type=jnp.float32)
    h = jax.nn.relu(h + bc_ref[...])
    cm = (cid > 0).astype(jnp.float32)[:, :, None]
    h = h - (1.0 - cm) * _NEG
    ch = (jnp.max(h, axis=1) * tm_ref[...]).astype(jnp.bfloat16)

    iota_p = jax.lax.broadcasted_iota(jnp.int32, (1, 64), 1)
    oh_p = (ids[:, _CLEN:_CLEN + 1] == iota_p).astype(jnp.bfloat16)

    gx_ref[...] = (
        jnp.dot(wx_ref[...].astype(jnp.bfloat16), ww_ref[...],
                preferred_element_type=jnp.float32)
        + jnp.dot(ch, wch_ref[...], preferred_element_type=jnp.float32)
        + jnp.dot(oh_p, mpos_ref[...], preferred_element_type=jnp.float32)
        + b_ref[...])


def _char_encode_project(ids, word_x, tok_mask, ce_pad,
                         wc_flat, bc, ww, wch, mpos, b, rt=512):
    n = ids.shape[0]
    f = ww.shape[1]
    nt = n // rt
    return pl.pallas_call(
        _encode_proj_kernel,
        out_shape=jax.ShapeDtypeStruct((n, f), jnp.float32),
        grid_spec=pltpu.PrefetchScalarGridSpec(
            num_scalar_prefetch=0,
            grid=(2, nt // 2),
            in_specs=[
                pl.BlockSpec((rt, 17), lambda c, i: (c * (nt // 2) + i, 0)),
                pl.BlockSpec((rt, word_x.shape[1]), lambda c, i: (c * (nt // 2) + i, 0)),
                pl.BlockSpec((rt, 1), lambda c, i: (c * (nt // 2) + i, 0)),
                pl.BlockSpec((128, _CE), lambda c, i: (0, 0)),
                pl.BlockSpec((3 * _CE, _HC), lambda c, i: (0, 0)),
                pl.BlockSpec((1, _HC), lambda c, i: (0, 0)),
                pl.BlockSpec((128, f), lambda c, i: (0, 0)),
                pl.BlockSpec((_HC, f), lambda c, i: (0, 0)),
                pl.BlockSpec((64, f), lambda c, i: (0, 0)),
                pl.BlockSpec((1, f), lambda c, i: (0, 0)),
            ],
            out_specs=pl.BlockSpec((rt, f), lambda c, i: (c * (nt // 2) + i, 0)),
        ),
        compiler_params=pltpu.CompilerParams(
            dimension_semantics=("parallel", "arbitrary")),
    )(ids, word_x, tok_mask, ce_pad, wc_flat, bc, ww, wch, mpos, b)


def _linear_kernel(x_ref, w_ref, b_ref, o_ref):
    o_ref[...] = (jnp.dot(x_ref[...].astype(jnp.bfloat16), w_ref[...],
                          preferred_element_type=jnp.float32) + b_ref[...])


def _row_linear(x, w, b, rt=512):
    n, d = x.shape
    f = w.shape[1]
    nt = n // rt
    return pl.pallas_call(
        _linear_kernel,
        out_shape=jax.ShapeDtypeStruct((n, f), jnp.float32),
        grid_spec=pltpu.PrefetchScalarGridSpec(
            num_scalar_prefetch=0,
            grid=(2, nt // 2),
            in_specs=[pl.BlockSpec((rt, d), lambda c, i: (c * (nt // 2) + i, 0)),
                      pl.BlockSpec((d, f), lambda c, i: (0, 0)),
                      pl.BlockSpec((1, f), lambda c, i: (0, 0))],
            out_specs=pl.BlockSpec((rt, f), lambda c, i: (c * (nt // 2) + i, 0)),
        ),
        compiler_params=pltpu.CompilerParams(
            dimension_semantics=("parallel", "arbitrary")),
    )(x, w, b)


def _lstm_kernel(gx_ref, m_ref, whh_ref, out_ref, h_scr, c_scr):
    d = pl.program_id(0)

    @pl.when(pl.program_id(1) == 0)
    def _():
        h_scr[...] = jnp.zeros_like(h_scr)
        c_scr[...] = jnp.zeros_like(c_scr)

    whh = whh_ref[0]
    for j in range(_TC):
        r = j + (_TC - 1 - 2 * j) * d
        sl = pl.ds(r, 1)
        gx_t = gx_ref[sl][0]
        m_t = m_ref[sl][0]
        h_prev = h_scr[...]
        c_prev = c_scr[...]

        gates = gx_t + jnp.dot(h_prev.astype(jnp.bfloat16), whh,
                               preferred_element_type=jnp.float32)
        i_g = jax.nn.sigmoid(gates[:, 0 * _H:1 * _H])
        f_g = jax.nn.sigmoid(gates[:, 1 * _H:2 * _H])
        g_g = jnp.tanh(gates[:, 2 * _H:3 * _H])
        o_g = jax.nn.sigmoid(gates[:, 3 * _H:4 * _H])

        c_new = f_g * c_prev + i_g * g_g
        h_new = o_g * jnp.tanh(c_new)

        valid = m_t > 0.0
        c_scr[...] = jnp.where(valid, c_new, c_prev)
        h_scr[...] = jnp.where(valid, h_new, h_prev)
        out_ref[sl] = (h_scr[...] * m_t)[None]


def _bilstm_layer(gx, mask_tb1, whh2):
    n_chunks = _T // _TC

    def t_in(c, i):
        return c * (n_chunks - 1) + (1 - 2 * c) * i

    return pl.pallas_call(
        _lstm_kernel,
        out_shape=jax.ShapeDtypeStruct((_T, _B, 2 * _H), jnp.float32),
        grid_spec=pltpu.PrefetchScalarGridSpec(
            num_scalar_prefetch=0,
            grid=(2, n_chunks),
            in_specs=[
                pl.BlockSpec((_TC, _B, 4 * _H), lambda c, i: (t_in(c, i), 0, c)),
                pl.BlockSpec((_TC, _B, 1), lambda c, i: (t_in(c, i), 0, 0)),
                pl.BlockSpec((1, _H, 4 * _H), lambda c, i: (c, 0, 0)),
            ],
            out_specs=pl.BlockSpec((_TC, _B, _H), lambda c, i: (t_in(c, i), 0, c)),
            scratch_shapes=[pltpu.VMEM((_B, _H), jnp.float32),
                            pltpu.VMEM((_B, _H), jnp.float32)],
        ),
        compiler_params=pltpu.CompilerParams(
            dimension_semantics=("parallel", "arbitrary")),
    )(gx, mask_tb1, whh2)


def _head_kernel(hid_ref, m_ref, hsse_ref, w1tok_ref, w1sent_ref, w1sse_ref,
                 b1_ref, w2_ref, b2_ref, out_ref):
    hid = hid_ref[...]
    m = m_ref[...]

    sent = jnp.max(hid - (1.0 - m) * _NEG, axis=0).astype(jnp.bfloat16)
    bias = (jnp.dot(sent, w1sent_ref[...], preferred_element_type=jnp.float32)
            + jnp.dot(hsse_ref[...].astype(jnp.bfloat16), w1sse_ref[...],
                      preferred_element_type=jnp.float32)
            + b1_ref[...])

    h1 = jax.lax.dot_general(hid.astype(jnp.bfloat16), w1tok_ref[...],
                             dimension_numbers=(((2,), (0,)), ((), ())),
                             preferred_element_type=jnp.float32)
    h1 = jax.nn.relu(h1 + bias[None]).astype(jnp.bfloat16)
    out_ref[...] = (jax.lax.dot_general(h1, w2_ref[...],
                                        dimension_numbers=(((2,), (0,)), ((), ())),
                                        preferred_element_type=jnp.float32)
                    + b2_ref[...])


def _fused_heads(hidden_tbd, mask_tb1, hsse, subj_w1, subj_b1, subj_w2,
                 subj_b2, obj_w1, obj_b1, obj_w2, obj_b2, bb=8):
    H2 = 2 * _H
    w1s_tok, w1s_sent = subj_w1[:H2], subj_w1[H2:]
    w1o_tok = obj_w1[:H2]
    w1o_sent = obj_w1[H2:2 * H2]
    w1o_sse = obj_w1[2 * H2:]

    w1_tok = jnp.concatenate([w1s_tok, w1o_tok], axis=1).astype(jnp.bfloat16)
    w1_sent = jnp.concatenate([w1s_sent, w1o_sent], axis=1).astype(jnp.bfloat16)
    w1_sse = jnp.concatenate(
        [jnp.zeros((2 * H2, H2), jnp.float32), w1o_sse],
        axis=1).astype(jnp.bfloat16)
    b1 = jnp.concatenate([subj_b1, obj_b1], axis=1)
    w2 = jnp.concatenate(
        [jnp.concatenate([subj_w2, jnp.zeros((H2, 2), jnp.float32)], axis=1),
         jnp.concatenate([jnp.zeros((H2, 2), jnp.float32), obj_w2], axis=1)],
        axis=0).astype(jnp.bfloat16)
    b2 = jnp.concatenate([subj_b2, obj_b2], axis=1)

    nb = _B // bb
    return pl.pallas_call(
        _head_kernel,
        out_shape=jax.ShapeDtypeStruct((_T, _B, 4), jnp.float32),
        grid_spec=pltpu.PrefetchScalarGridSpec(
            num_scalar_prefetch=0,
            grid=(2, nb // 2),
            in_specs=[
                pl.BlockSpec((_T, bb, H2), lambda c, i: (0, c * (nb // 2) + i, 0)),
                pl.BlockSpec((_T, bb, 1), lambda c, i: (0, c * (nb // 2) + i, 0)),
                pl.BlockSpec((bb, 2 * H2), lambda c, i: (c * (nb // 2) + i, 0)),
                pl.BlockSpec((H2, 2 * H2), lambda c, i: (0, 0)),
                pl.BlockSpec((H2, 2 * H2), lambda c, i: (0, 0)),
                pl.BlockSpec((2 * H2, 2 * H2), lambda c, i: (0, 0)),
                pl.BlockSpec((1, 2 * H2), lambda c, i: (0, 0)),
                pl.BlockSpec((2 * H2, 4), lambda c, i: (0, 0)),
                pl.BlockSpec((1, 4), lambda c, i: (0, 0)),
            ],
            out_specs=pl.BlockSpec((_T, bb, 4), lambda c, i: (0, c * (nb // 2) + i, 0)),
        ),
        compiler_params=pltpu.CompilerParams(
            dimension_semantics=("parallel", "arbitrary")),
    )(hidden_tbd, mask_tb1, hsse, w1_tok, w1_sent, w1_sse, b1, w2, b2)


def kernel(words, chars, pos_tags, subj_start_position, subj_end_position, mask,
           nearest_subj_position_for_each_token, distance_to_nearest_subj,
           distance_to_subj, nearest_obj_start_position_for_each_token,
           distance_to_nearest_obj_start,
           word_emb, char_emb, pos_emb, char_conv_w, char_conv_b,
           l0_fwd_wih, l0_fwd_whh, l0_fwd_b, l0_bwd_wih, l0_bwd_whh, l0_bwd_b,
           l1_fwd_wih, l1_fwd_whh, l1_fwd_b, l1_bwd_wih, l1_bwd_whh, l1_bwd_b,
           subj_w1, subj_b1, subj_w2, subj_b2, obj_w1, obj_b1, obj_w2, obj_b2):
    n = _T * _B

    words_t = words.T
    pos_t = pos_tags.T
    chars_t = jnp.transpose(chars, (1, 0, 2)).reshape(n, _CLEN)
    ids = jnp.concatenate([chars_t, pos_t.reshape(n, 1)], axis=1)
    mask_t = mask.T
    tok_mask = mask_t.reshape(n, 1)

    word_x = jnp.take(word_emb, words_t.reshape(n), axis=0)

    wih0 = jnp.concatenate([l0_fwd_wih, l0_bwd_wih], axis=1)
    b0 = jnp.concatenate([l0_fwd_b, l0_bwd_b], axis=1)
    ww = wih0[:128].astype(jnp.bfloat16)
    wch = wih0[128:256].astype(jnp.bfloat16)
    mpos = jnp.zeros((64, 8 * _H), jnp.float32)
    mpos = mpos.at[:50].set(pos_emb @ wih0[256:384]).astype(jnp.bfloat16)
    ce_pad = jnp.zeros((128, _CE), jnp.float32).at[:100].set(
        char_emb).astype(jnp.bfloat16)
    wc_flat = char_conv_w.reshape(3 * _CE, _HC).astype(jnp.bfloat16)
    gx0 = _char_encode_project(ids, word_x, tok_mask, ce_pad,
                               wc_flat, char_conv_b, ww, wch, mpos, b0)
    gx0 = gx0.reshape(_T, _B, 8 * _H)
    mask_tb1 = mask_t.reshape(_T, _B, 1)
    whh0 = jnp.stack([l0_fwd_whh, l0_bwd_whh],
                     axis=0).astype(jnp.bfloat16)
    h0 = _bilstm_layer(gx0, mask_tb1, whh0)

    wih1 = jnp.concatenate([l1_fwd_wih, l1_bwd_wih],
                           axis=1).astype(jnp.bfloat16)
    b1 = jnp.concatenate([l1_fwd_b, l1_bwd_b], axis=1)
    gx1 = _row_linear(h0.reshape(n, 2 * _H), wih1, b1).reshape(_T, _B, 8 * _H)
    whh1 = jnp.stack([l1_fwd_whh, l1_bwd_whh], axis=0).astype(jnp.bfloat16)
    h1 = _bilstm_layer(gx1, mask_tb1, whh1)

    bidx = jnp.arange(_B)
    h_ss = h1[subj_start_position, bidx]
    h_se = h1[subj_end_position, bidx]
    hsse = jnp.concatenate([h_ss, h_se], axis=1)
    logits = _fused_heads(h1, mask_tb1, hsse, subj_w1, subj_b1, subj_w2,
                          subj_b2, obj_w1, obj_b1, obj_w2, obj_b2)
    logits = jnp.transpose(logits, (1, 0, 2))
    return (logits[:, :, 0], logits[:, :, 1], logits[:, :, 2], logits[:, :, 3])


# speedup vs baseline: 1.1101x; 1.1101x over previous
import jax
import jax.numpy as jnp
from jax.experimental import pallas as pl
from jax.experimental.pallas import tpu as pltpu


_H = 256
_B = 32
_T = 128
_CLEN = 16
_CE = 64
_HC = 128
_TC = 8
_NEG = 1e10


def _encode_proj_kernel(ids_ref, wx_ref, tm_ref, ce_ref,
                        wc_ref, bc_ref, ww_ref, wch_ref, mpos_ref, b_ref,
                        gx_ref):
    ids = ids_ref[...]
    cid = ids[:, :_CLEN]
    rt = cid.shape[0]

    iota_c = jax.lax.broadcasted_iota(jnp.int32, (1, 1, 128), 2)
    oh_c = (cid[:, :, None] == iota_c).astype(jnp.bfloat16)
    x = jax.lax.dot_general(oh_c, ce_ref[...],
                            dimension_numbers=(((2,), (0,)), ((), ())),
                            preferred_element_type=jnp.float32)
    x = x.astype(jnp.bfloat16)

    z = jnp.zeros((rt, 1, _CE), x.dtype)
    patches = jnp.concatenate(
        [jnp.concatenate([z, x[:, :-1, :]], axis=1),
         x,
         jnp.concatenate([x[:, 1:, :], z], axis=1)], axis=2)
    h = jax.lax.dot_general(patches, wc_ref[...],
                            dimension_numbers=(((2,), (0,)), ((), ())),
                            preferred_element_type=jnp.float32)
    h = jax.nn.relu(h + bc_ref[...])
    cm = (cid > 0).astype(jnp.float32)[:, :, None]
    h = h - (1.0 - cm) * _NEG
    ch = (jnp.max(h, axis=1) * tm_ref[...]).astype(jnp.bfloat16)

    iota_p = jax.lax.broadcasted_iota(jnp.int32, (1, 64), 1)
    oh_p = (ids[:, _CLEN:_CLEN + 1] == iota_p).astype(jnp.bfloat16)

    gx_ref[...] = (
        jnp.dot(wx_ref[...].astype(jnp.bfloat16), ww_ref[...],
                preferred_element_type=jnp.float32)
        + jnp.dot(ch, wch_ref[...], preferred_element_type=jnp.float32)
        + jnp.dot(oh_p, mpos_ref[...], preferred_element_type=jnp.float32)
        + b_ref[...])


def _char_encode_project(ids, word_x, tok_mask, ce_pad,
                         wc_flat, bc, ww, wch, mpos, b, rt=512):
    n = ids.shape[0]
    f = ww.shape[1]
    nt = n // rt
    return pl.pallas_call(
        _encode_proj_kernel,
        out_shape=jax.ShapeDtypeStruct((n, f), jnp.float32),
        grid_spec=pltpu.PrefetchScalarGridSpec(
            num_scalar_prefetch=0,
            grid=(2, nt // 2),
            in_specs=[
                pl.BlockSpec((rt, 17), lambda c, i: (c * (nt // 2) + i, 0)),
                pl.BlockSpec((rt, word_x.shape[1]), lambda c, i: (c * (nt // 2) + i, 0)),
                pl.BlockSpec((rt, 1), lambda c, i: (c * (nt // 2) + i, 0)),
                pl.BlockSpec((128, _CE), lambda c, i: (0, 0)),
                pl.BlockSpec((3 * _CE, _HC), lambda c, i: (0, 0)),
                pl.BlockSpec((1, _HC), lambda c, i: (0, 0)),
                pl.BlockSpec((128, f), lambda c, i: (0, 0)),
                pl.BlockSpec((_HC, f), lambda c, i: (0, 0)),
                pl.BlockSpec((64, f), lambda c, i: (0, 0)),
                pl.BlockSpec((1, f), lambda c, i: (0, 0)),
            ],
            out_specs=pl.BlockSpec((rt, f), lambda c, i: (c * (nt // 2) + i, 0)),
        ),
        compiler_params=pltpu.CompilerParams(
            dimension_semantics=("parallel", "arbitrary")),
    )(ids, word_x, tok_mask, ce_pad, wc_flat, bc, ww, wch, mpos, b)


def _linear2_kernel(xf_ref, xb_ref, wf_ref, wb_ref, b_ref, o_ref):
    o_ref[...] = (
        jnp.dot(xf_ref[...].astype(jnp.bfloat16), wf_ref[...],
                preferred_element_type=jnp.float32)
        + jnp.dot(xb_ref[...].astype(jnp.bfloat16), wb_ref[...],
                  preferred_element_type=jnp.float32)
        + b_ref[...])


def _row_linear2(xf, xb, wf, wb, b, rt=512):
    n, d = xf.shape
    f = wf.shape[1]
    nt = n // rt
    return pl.pallas_call(
        _linear2_kernel,
        out_shape=jax.ShapeDtypeStruct((n, f), jnp.float32),
        grid_spec=pltpu.PrefetchScalarGridSpec(
            num_scalar_prefetch=0,
            grid=(2, nt // 2),
            in_specs=[pl.BlockSpec((rt, d), lambda c, i: (c * (nt // 2) + i, 0)),
                      pl.BlockSpec((rt, d), lambda c, i: (c * (nt // 2) + i, 0)),
                      pl.BlockSpec((d, f), lambda c, i: (0, 0)),
                      pl.BlockSpec((d, f), lambda c, i: (0, 0)),
                      pl.BlockSpec((1, f), lambda c, i: (0, 0))],
            out_specs=pl.BlockSpec((rt, f), lambda c, i: (c * (nt // 2) + i, 0)),
        ),
        compiler_params=pltpu.CompilerParams(
            dimension_semantics=("parallel", "arbitrary")),
    )(xf, xb, wf, wb, b)


def _lstm_kernel(gxf_ref, gxb_ref, m_ref, mb_ref, whf_ref, whb_ref,
                 hf_ref, hb_ref, hf_scr, cf_scr, hb_scr, cb_scr):
    @pl.when(pl.program_id(0) == 0)
    def _():
        hf_scr[...] = jnp.zeros_like(hf_scr)
        cf_scr[...] = jnp.zeros_like(cf_scr)
        hb_scr[...] = jnp.zeros_like(hb_scr)
        cb_scr[...] = jnp.zeros_like(cb_scr)

    whf = whf_ref[...]
    whb = whb_ref[...]

    def step(gates, c_prev, h_prev, m_t):
        i_g = jax.nn.sigmoid(gates[:, 0 * _H:1 * _H])
        f_g = jax.nn.sigmoid(gates[:, 1 * _H:2 * _H])
        g_g = jnp.tanh(gates[:, 2 * _H:3 * _H])
        o_g = jax.nn.sigmoid(gates[:, 3 * _H:4 * _H])
        c_new = f_g * c_prev + i_g * g_g
        h_new = o_g * jnp.tanh(c_new)
        valid = m_t > 0.0
        return jnp.where(valid, c_new, c_prev), jnp.where(valid, h_new, h_prev)

    for j in range(_TC):
        jb = _TC - 1 - j
        gates_f = gxf_ref[j] + jnp.dot(hf_scr[...].astype(jnp.bfloat16), whf,
                                       preferred_element_type=jnp.float32)
        gates_b = gxb_ref[jb] + jnp.dot(hb_scr[...].astype(jnp.bfloat16), whb,
                                        preferred_element_type=jnp.float32)
        mf_t = m_ref[j]
        mb_t = mb_ref[jb]
        cf, hf = step(gates_f, cf_scr[...], hf_scr[...], mf_t)
        cb, hb = step(gates_b, cb_scr[...], hb_scr[...], mb_t)
        cf_scr[...] = cf
        hf_scr[...] = hf
        cb_scr[...] = cb
        hb_scr[...] = hb
        hf_ref[j] = hf * mf_t
        hb_ref[jb] = hb * mb_t


def _bilstm_layer(gx, mask_tb1, whh_f, whh_b):
    n_chunks = _T // _TC
    rev = n_chunks - 1
    out = jax.ShapeDtypeStruct((_T, _B, _H), jnp.float32)
    return pl.pallas_call(
        _lstm_kernel,
        out_shape=[out, out],
        grid_spec=pltpu.PrefetchScalarGridSpec(
            num_scalar_prefetch=0,
            grid=(n_chunks,),
            in_specs=[
                pl.BlockSpec((_TC, _B, 4 * _H), lambda i: (i, 0, 0)),
                pl.BlockSpec((_TC, _B, 4 * _H), lambda i: (rev - i, 0, 1)),
                pl.BlockSpec((_TC, _B, 1), lambda i: (i, 0, 0)),
                pl.BlockSpec((_TC, _B, 1), lambda i: (rev - i, 0, 0)),
                pl.BlockSpec((_H, 4 * _H), lambda i: (0, 0)),
                pl.BlockSpec((_H, 4 * _H), lambda i: (0, 0)),
            ],
            out_specs=[pl.BlockSpec((_TC, _B, _H), lambda i: (i, 0, 0)),
                       pl.BlockSpec((_TC, _B, _H), lambda i: (rev - i, 0, 0))],
            scratch_shapes=[pltpu.VMEM((_B, _H), jnp.float32)] * 4,
        ),
        compiler_params=pltpu.CompilerParams(
            dimension_semantics=("arbitrary",)),
    )(gx, gx, mask_tb1, mask_tb1, whh_f, whh_b)


def _head_kernel(hf_ref, hb_ref, m_ref, hsse_ref, w1tok_ref, w1sent_ref,
                 w1sse_ref, b1_ref, w2_ref, b2_ref, out_ref):
    hid = jnp.concatenate([hf_ref[...], hb_ref[...]], axis=2)
    m = m_ref[...]

    sent = jnp.max(hid - (1.0 - m) * _NEG, axis=0).astype(jnp.bfloat16)
    bias = (jnp.dot(sent, w1sent_ref[...], preferred_element_type=jnp.float32)
            + jnp.dot(hsse_ref[...].astype(jnp.bfloat16), w1sse_ref[...],
                      preferred_element_type=jnp.float32)
            + b1_ref[...])

    h1 = jax.lax.dot_general(hid.astype(jnp.bfloat16), w1tok_ref[...],
                             dimension_numbers=(((2,), (0,)), ((), ())),
                             preferred_element_type=jnp.float32)
    h1 = jax.nn.relu(h1 + bias[None]).astype(jnp.bfloat16)
    out_ref[...] = (jax.lax.dot_general(h1, w2_ref[...],
                                        dimension_numbers=(((2,), (0,)), ((), ())),
                                        preferred_element_type=jnp.float32)
                    + b2_ref[...])


def _fused_heads(hf, hb, mask_tb1, hsse, subj_w1, subj_b1, subj_w2,
                 subj_b2, obj_w1, obj_b1, obj_w2, obj_b2, bb=8):
    H2 = 2 * _H
    w1s_tok, w1s_sent = subj_w1[:H2], subj_w1[H2:]
    w1o_tok = obj_w1[:H2]
    w1o_sent = obj_w1[H2:2 * H2]
    w1o_sse = obj_w1[2 * H2:]

    w1_tok = jnp.concatenate([w1s_tok, w1o_tok], axis=1).astype(jnp.bfloat16)
    w1_sent = jnp.concatenate([w1s_sent, w1o_sent], axis=1).astype(jnp.bfloat16)
    w1_sse = jnp.concatenate(
        [jnp.zeros((2 * H2, H2), jnp.float32), w1o_sse],
        axis=1).astype(jnp.bfloat16)
    b1 = jnp.concatenate([subj_b1, obj_b1], axis=1)
    w2 = jnp.concatenate(
        [jnp.concatenate([subj_w2, jnp.zeros((H2, 2), jnp.float32)], axis=1),
         jnp.concatenate([jnp.zeros((H2, 2), jnp.float32), obj_w2], axis=1)],
        axis=0).astype(jnp.bfloat16)
    b2 = jnp.concatenate([subj_b2, obj_b2], axis=1)

    nb = _B // bb
    return pl.pallas_call(
        _head_kernel,
        out_shape=jax.ShapeDtypeStruct((_T, _B, 4), jnp.float32),
        grid_spec=pltpu.PrefetchScalarGridSpec(
            num_scalar_prefetch=0,
            grid=(2, nb // 2),
            in_specs=[
                pl.BlockSpec((_T, bb, _H), lambda c, i: (0, c * (nb // 2) + i, 0)),
                pl.BlockSpec((_T, bb, _H), lambda c, i: (0, c * (nb // 2) + i, 0)),
                pl.BlockSpec((_T, bb, 1), lambda c, i: (0, c * (nb // 2) + i, 0)),
                pl.BlockSpec((bb, 2 * H2), lambda c, i: (c * (nb // 2) + i, 0)),
                pl.BlockSpec((H2, 2 * H2), lambda c, i: (0, 0)),
                pl.BlockSpec((H2, 2 * H2), lambda c, i: (0, 0)),
                pl.BlockSpec((2 * H2, 2 * H2), lambda c, i: (0, 0)),
                pl.BlockSpec((1, 2 * H2), lambda c, i: (0, 0)),
                pl.BlockSpec((2 * H2, 4), lambda c, i: (0, 0)),
                pl.BlockSpec((1, 4), lambda c, i: (0, 0)),
            ],
            out_specs=pl.BlockSpec((_T, bb, 4), lambda c, i: (0, c * (nb // 2) + i, 0)),
        ),
        compiler_params=pltpu.CompilerParams(
            dimension_semantics=("parallel", "arbitrary")),
    )(hf, hb, mask_tb1, hsse, w1_tok, w1_sent, w1_sse, b1, w2, b2)


def kernel(words, chars, pos_tags, subj_start_position, subj_end_position, mask,
           nearest_subj_position_for_each_token, distance_to_nearest_subj,
           distance_to_subj, nearest_obj_start_position_for_each_token,
           distance_to_nearest_obj_start,
           word_emb, char_emb, pos_emb, char_conv_w, char_conv_b,
           l0_fwd_wih, l0_fwd_whh, l0_fwd_b, l0_bwd_wih, l0_bwd_whh, l0_bwd_b,
           l1_fwd_wih, l1_fwd_whh, l1_fwd_b, l1_bwd_wih, l1_bwd_whh, l1_bwd_b,
           subj_w1, subj_b1, subj_w2, subj_b2, obj_w1, obj_b1, obj_w2, obj_b2):
    n = _T * _B

    words_t = words.T
    pos_t = pos_tags.T
    chars_t = jnp.transpose(chars, (1, 0, 2)).reshape(n, _CLEN)
    ids = jnp.concatenate([chars_t, pos_t.reshape(n, 1)], axis=1)
    mask_t = mask.T
    tok_mask = mask_t.reshape(n, 1)

    word_x = jnp.take(word_emb, words_t.reshape(n), axis=0)

    wih0 = jnp.concatenate([l0_fwd_wih, l0_bwd_wih], axis=1)
    b0 = jnp.concatenate([l0_fwd_b, l0_bwd_b], axis=1)
    ww = wih0[:128].astype(jnp.bfloat16)
    wch = wih0[128:256].astype(jnp.bfloat16)
    mpos = jnp.zeros((64, 8 * _H), jnp.float32)
    mpos = mpos.at[:50].set(pos_emb @ wih0[256:384]).astype(jnp.bfloat16)
    ce_pad = jnp.zeros((128, _CE), jnp.float32).at[:100].set(
        char_emb).astype(jnp.bfloat16)
    wc_flat = char_conv_w.reshape(3 * _CE, _HC).astype(jnp.bfloat16)
    gx0 = _char_encode_project(ids, word_x, tok_mask, ce_pad,
                               wc_flat, char_conv_b, ww, wch, mpos, b0)
    gx0 = gx0.reshape(_T, _B, 8 * _H)
    mask_tb1 = mask_t.reshape(_T, _B, 1)
    h0f, h0b = _bilstm_layer(gx0, mask_tb1,
                             l0_fwd_whh.astype(jnp.bfloat16),
                             l0_bwd_whh.astype(jnp.bfloat16))

    wih1 = jnp.concatenate([l1_fwd_wih, l1_bwd_wih],
                           axis=1).astype(jnp.bfloat16)
    b1 = jnp.concatenate([l1_fwd_b, l1_bwd_b], axis=1)
    gx1 = _row_linear2(h0f.reshape(n, _H), h0b.reshape(n, _H),
                       wih1[:_H], wih1[_H:], b1).reshape(_T, _B, 8 * _H)
    h1f, h1b = _bilstm_layer(gx1, mask_tb1,
                             l1_fwd_whh.astype(jnp.bfloat16),
                             l1_bwd_whh.astype(jnp.bfloat16))

    bidx = jnp.arange(_B)
    hsse = jnp.concatenate(
        [h1f[subj_start_position, bidx], h1b[subj_start_position, bidx],
         h1f[subj_end_position, bidx], h1b[subj_end_position, bidx]],
        axis=1)
    logits = _fused_heads(h1f, h1b, mask_tb1, hsse, subj_w1, subj_b1, subj_w2,
                          subj_b2, obj_w1, obj_b1, obj_w2, obj_b2)
    logits = jnp.transpose(logits, (1, 0, 2))
    return (logits[:, :, 0], logits[:, :, 1], logits[:, :, 2], logits[:, :, 3])


# speedup vs baseline: 1.2800x; 1.1530x over previous
import jax
import jax.numpy as jnp
from jax.experimental import pallas as pl
from jax.experimental.pallas import tpu as pltpu


_H = 256
_B = 32
_T = 128
_CLEN = 16
_CE = 64
_HC = 128
_TC = 8
_NEG = 1e10


def _char_encode_kernel(ids_ref, tm_ref, mcomb_ref, bc_ref, ch_ref):
    cid = ids_ref[...][:, :_CLEN]
    rt = cid.shape[0]

    iota_c = jax.lax.broadcasted_iota(jnp.int32, (1, 1, 128), 2)
    oh_c = (cid[:, :, None] == iota_c).astype(jnp.bfloat16)
    hall = jax.lax.dot_general(oh_c, mcomb_ref[...],
                               dimension_numbers=(((2,), (0,)), ((), ())),
                               preferred_element_type=jnp.float32)
    a = hall[:, :, :_HC]
    bmid = hall[:, :, _HC:2 * _HC]
    c = hall[:, :, 2 * _HC:]
    z = jnp.zeros((rt, 1, _HC), jnp.float32)
    conv = (bmid
            + jnp.concatenate([z, a[:, :-1, :]], axis=1)
            + jnp.concatenate([c[:, 1:, :], z], axis=1))
    h = jax.nn.relu(conv + bc_ref[...])
    cm = (cid > 0).astype(jnp.float32)[:, :, None]
    h = h - (1.0 - cm) * _NEG
    ch_ref[...] = (jnp.max(h, axis=1) * tm_ref[...]).astype(jnp.bfloat16)


def _char_encode(ids, tok_mask, mcomb, bc, rt=256):
    n = ids.shape[0]
    nt = n // rt
    return pl.pallas_call(
        _char_encode_kernel,
        out_shape=jax.ShapeDtypeStruct((n, _HC), jnp.bfloat16),
        grid_spec=pltpu.PrefetchScalarGridSpec(
            num_scalar_prefetch=0,
            grid=(2, nt // 2),
            in_specs=[
                pl.BlockSpec((rt, 17), lambda c, i: (c * (nt // 2) + i, 0)),
                pl.BlockSpec((rt, 1), lambda c, i: (c * (nt // 2) + i, 0)),
                pl.BlockSpec((128, 3 * _HC), lambda c, i: (0, 0)),
                pl.BlockSpec((1, _HC), lambda c, i: (0, 0)),
            ],
            out_specs=pl.BlockSpec((rt, _HC), lambda c, i: (c * (nt // 2) + i, 0)),
        ),
        compiler_params=pltpu.CompilerParams(
            dimension_semantics=("parallel", "arbitrary")),
    )(ids, tok_mask, mcomb, bc)


def _proj0_kernel(ids_ref, wx_ref, ch_ref, ww_ref, wch_ref, mpos_ref, b_ref,
                  gx_ref):
    iota_p = jax.lax.broadcasted_iota(jnp.int32, (1, 64), 1)
    oh_p = (ids_ref[...][:, _CLEN:_CLEN + 1] == iota_p).astype(jnp.bfloat16)

    gx_ref[...] = (
        jnp.dot(wx_ref[...].astype(jnp.bfloat16), ww_ref[...],
                preferred_element_type=jnp.float32)
        + jnp.dot(ch_ref[...], wch_ref[...], preferred_element_type=jnp.float32)
        + jnp.dot(oh_p, mpos_ref[...], preferred_element_type=jnp.float32)
        + b_ref[...]).astype(jnp.bfloat16)


def _proj0(ids, word_x, ch, ww, wch, mpos, b, rt=512):
    n = ids.shape[0]
    f = ww.shape[1]
    nt = n // rt
    return pl.pallas_call(
        _proj0_kernel,
        out_shape=jax.ShapeDtypeStruct((n, f), jnp.bfloat16),
        grid_spec=pltpu.PrefetchScalarGridSpec(
            num_scalar_prefetch=0,
            grid=(2, nt // 2),
            in_specs=[
                pl.BlockSpec((rt, 17), lambda c, i: (c * (nt // 2) + i, 0)),
                pl.BlockSpec((rt, word_x.shape[1]), lambda c, i: (c * (nt // 2) + i, 0)),
                pl.BlockSpec((rt, _HC), lambda c, i: (c * (nt // 2) + i, 0)),
                pl.BlockSpec((128, f), lambda c, i: (0, 0)),
                pl.BlockSpec((_HC, f), lambda c, i: (0, 0)),
                pl.BlockSpec((64, f), lambda c, i: (0, 0)),
                pl.BlockSpec((1, f), lambda c, i: (0, 0)),
            ],
            out_specs=pl.BlockSpec((rt, f), lambda c, i: (c * (nt // 2) + i, 0)),
        ),
        compiler_params=pltpu.CompilerParams(
            dimension_semantics=("parallel", "arbitrary")),
    )(ids, word_x, ch, ww, wch, mpos, b)


def _linear2_kernel(xf_ref, xb_ref, wf_ref, wb_ref, b_ref, o_ref):
    o_ref[...] = (
        jnp.dot(xf_ref[...].astype(jnp.bfloat16), wf_ref[...],
                preferred_element_type=jnp.float32)
        + jnp.dot(xb_ref[...].astype(jnp.bfloat16), wb_ref[...],
                  preferred_element_type=jnp.float32)
        + b_ref[...]).astype(jnp.bfloat16)


def _row_linear2(xf, xb, wf, wb, b, rt=512):
    n, d = xf.shape
    f = wf.shape[1]
    nt = n // rt
    return pl.pallas_call(
        _linear2_kernel,
        out_shape=jax.ShapeDtypeStruct((n, f), jnp.bfloat16),
        grid_spec=pltpu.PrefetchScalarGridSpec(
            num_scalar_prefetch=0,
            grid=(2, nt // 2),
            in_specs=[pl.BlockSpec((rt, d), lambda c, i: (c * (nt // 2) + i, 0)),
                      pl.BlockSpec((rt, d), lambda c, i: (c * (nt // 2) + i, 0)),
                      pl.BlockSpec((d, f), lambda c, i: (0, 0)),
                      pl.BlockSpec((d, f), lambda c, i: (0, 0)),
                      pl.BlockSpec((1, f), lambda c, i: (0, 0))],
            out_specs=pl.BlockSpec((rt, f), lambda c, i: (c * (nt // 2) + i, 0)),
        ),
        compiler_params=pltpu.CompilerParams(
            dimension_semantics=("parallel", "arbitrary")),
    )(xf, xb, wf, wb, b)


def _lstm_kernel(gxf_ref, gxb_ref, m_ref, mb_ref, whf_ref, whb_ref,
                 hf_ref, hb_ref, hf_scr, cf_scr, hb_scr, cb_scr):
    @pl.when(pl.program_id(0) == 0)
    def _():
        hf_scr[...] = jnp.zeros_like(hf_scr)
        cf_scr[...] = jnp.zeros_like(cf_scr)
        hb_scr[...] = jnp.zeros_like(hb_scr)
        cb_scr[...] = jnp.zeros_like(cb_scr)

    whf = whf_ref[...]
    whb = whb_ref[...]

    def step(gates, c_prev, h_prev, m_t):
        i_g = jax.nn.sigmoid(gates[:, 0 * _H:1 * _H])
        f_g = jax.nn.sigmoid(gates[:, 1 * _H:2 * _H])
        g_g = jnp.tanh(gates[:, 2 * _H:3 * _H])
        o_g = jax.nn.sigmoid(gates[:, 3 * _H:4 * _H])
        c_new = f_g * c_prev + i_g * g_g
        h_new = o_g * jnp.tanh(c_new)
        valid = m_t > 0.0
        return jnp.where(valid, c_new, c_prev), jnp.where(valid, h_new, h_prev)

    for j in range(_TC):
        jb = _TC - 1 - j
        gates_f = gxf_ref[j] + jnp.dot(hf_scr[...].astype(jnp.bfloat16), whf,
                                       preferred_element_type=jnp.float32)
        gates_b = gxb_ref[jb] + jnp.dot(hb_scr[...].astype(jnp.bfloat16), whb,
                                        preferred_element_type=jnp.float32)
        mf_t = m_ref[j]
        mb_t = mb_ref[jb]
        cf, hf = step(gates_f, cf_scr[...], hf_scr[...], mf_t)
        cb, hb = step(gates_b, cb_scr[...], hb_scr[...], mb_t)
        cf_scr[...] = cf
        hf_scr[...] = hf
        cb_scr[...] = cb
        hb_scr[...] = hb
        hf_ref[j] = (hf * mf_t).astype(jnp.bfloat16)
        hb_ref[jb] = (hb * mb_t).astype(jnp.bfloat16)


def _bilstm_layer(gx, mask_tb1, whh_f, whh_b):
    n_chunks = _T // _TC
    rev = n_chunks - 1
    out = jax.ShapeDtypeStruct((_T, _B, _H), jnp.bfloat16)
    return pl.pallas_call(
        _lstm_kernel,
        out_shape=[out, out],
        grid_spec=pltpu.PrefetchScalarGridSpec(
            num_scalar_prefetch=0,
            grid=(n_chunks,),
            in_specs=[
                pl.BlockSpec((_TC, _B, 4 * _H), lambda i: (i, 0, 0)),
                pl.BlockSpec((_TC, _B, 4 * _H), lambda i: (rev - i, 0, 1)),
                pl.BlockSpec((_TC, _B, 1), lambda i: (i, 0, 0)),
                pl.BlockSpec((_TC, _B, 1), lambda i: (rev - i, 0, 0)),
                pl.BlockSpec((_H, 4 * _H), lambda i: (0, 0)),
                pl.BlockSpec((_H, 4 * _H), lambda i: (0, 0)),
            ],
            out_specs=[pl.BlockSpec((_TC, _B, _H), lambda i: (i, 0, 0)),
                       pl.BlockSpec((_TC, _B, _H), lambda i: (rev - i, 0, 0))],
            scratch_shapes=[pltpu.VMEM((_B, _H), jnp.float32)] * 4,
        ),
        compiler_params=pltpu.CompilerParams(
            dimension_semantics=("arbitrary",)),
    )(gx, gx, mask_tb1, mask_tb1, whh_f, whh_b)


def _head_kernel(hf_ref, hb_ref, m_ref, hsse_ref, w1tok_ref, w1sent_ref,
                 w1sse_ref, b1_ref, w2_ref, b2_ref, out_ref):
    hid = jnp.concatenate([hf_ref[...], hb_ref[...]], axis=2)
    m = m_ref[...]

    sent = jnp.max(hid.astype(jnp.float32) - (1.0 - m) * _NEG,
                   axis=0).astype(jnp.bfloat16)
    bias = (jnp.dot(sent, w1sent_ref[...], preferred_element_type=jnp.float32)
            + jnp.dot(hsse_ref[...].astype(jnp.bfloat16), w1sse_ref[...],
                      preferred_element_type=jnp.float32)
            + b1_ref[...])

    h1 = jax.lax.dot_general(hid.astype(jnp.bfloat16), w1tok_ref[...],
                             dimension_numbers=(((2,), (0,)), ((), ())),
                             preferred_element_type=jnp.float32)
    h1 = jax.nn.relu(h1 + bias[None]).astype(jnp.bfloat16)
    out_ref[...] = (jax.lax.dot_general(h1, w2_ref[...],
                                        dimension_numbers=(((2,), (0,)), ((), ())),
                                        preferred_element_type=jnp.float32)
                    + b2_ref[...])


def _fused_heads(hf, hb, mask_tb1, hsse, subj_w1, subj_b1, subj_w2,
                 subj_b2, obj_w1, obj_b1, obj_w2, obj_b2, bb=8):
    H2 = 2 * _H
    w1s_tok, w1s_sent = subj_w1[:H2], subj_w1[H2:]
    w1o_tok = obj_w1[:H2]
    w1o_sent = obj_w1[H2:2 * H2]
    w1o_sse = obj_w1[2 * H2:]

    w1_tok = jnp.concatenate([w1s_tok, w1o_tok], axis=1).astype(jnp.bfloat16)
    w1_sent = jnp.concatenate([w1s_sent, w1o_sent], axis=1).astype(jnp.bfloat16)
    w1_sse = jnp.concatenate(
        [jnp.zeros((2 * H2, H2), jnp.float32), w1o_sse],
        axis=1).astype(jnp.bfloat16)
    b1 = jnp.concatenate([subj_b1, obj_b1], axis=1)
    w2 = jnp.concatenate(
        [jnp.concatenate([subj_w2, jnp.zeros((H2, 2), jnp.float32)], axis=1),
         jnp.concatenate([jnp.zeros((H2, 2), jnp.float32), obj_w2], axis=1)],
        axis=0).astype(jnp.bfloat16)
    b2 = jnp.concatenate([subj_b2, obj_b2], axis=1)

    nb = _B // bb
    return pl.pallas_call(
        _head_kernel,
        out_shape=jax.ShapeDtypeStruct((_T, _B, 4), jnp.float32),
        grid_spec=pltpu.PrefetchScalarGridSpec(
            num_scalar_prefetch=0,
            grid=(2, nb // 2),
            in_specs=[
                pl.BlockSpec((_T, bb, _H), lambda c, i: (0, c * (nb // 2) + i, 0)),
                pl.BlockSpec((_T, bb, _H), lambda c, i: (0, c * (nb // 2) + i, 0)),
                pl.BlockSpec((_T, bb, 1), lambda c, i: (0, c * (nb // 2) + i, 0)),
                pl.BlockSpec((bb, 2 * H2), lambda c, i: (c * (nb // 2) + i, 0)),
                pl.BlockSpec((H2, 2 * H2), lambda c, i: (0, 0)),
                pl.BlockSpec((H2, 2 * H2), lambda c, i: (0, 0)),
                pl.BlockSpec((2 * H2, 2 * H2), lambda c, i: (0, 0)),
                pl.BlockSpec((1, 2 * H2), lambda c, i: (0, 0)),
                pl.BlockSpec((2 * H2, 4), lambda c, i: (0, 0)),
                pl.BlockSpec((1, 4), lambda c, i: (0, 0)),
            ],
            out_specs=pl.BlockSpec((_T, bb, 4), lambda c, i: (0, c * (nb // 2) + i, 0)),
        ),
        compiler_params=pltpu.CompilerParams(
            dimension_semantics=("parallel", "arbitrary")),
    )(hf, hb, mask_tb1, hsse, w1_tok, w1_sent, w1_sse, b1, w2, b2)


def kernel(words, chars, pos_tags, subj_start_position, subj_end_position, mask,
           nearest_subj_position_for_each_token, distance_to_nearest_subj,
           distance_to_subj, nearest_obj_start_position_for_each_token,
           distance_to_nearest_obj_start,
           word_emb, char_emb, pos_emb, char_conv_w, char_conv_b,
           l0_fwd_wih, l0_fwd_whh, l0_fwd_b, l0_bwd_wih, l0_bwd_whh, l0_bwd_b,
           l1_fwd_wih, l1_fwd_whh, l1_fwd_b, l1_bwd_wih, l1_bwd_whh, l1_bwd_b,
           subj_w1, subj_b1, subj_w2, subj_b2, obj_w1, obj_b1, obj_w2, obj_b2):
    n = _T * _B

    words_t = words.T
    pos_t = pos_tags.T
    chars_t = jnp.transpose(chars, (1, 0, 2)).reshape(n, _CLEN)
    ids = jnp.concatenate([chars_t, pos_t.reshape(n, 1)], axis=1)
    mask_t = mask.T
    tok_mask = mask_t.reshape(n, 1)

    word_x = jnp.take(word_emb, words_t.reshape(n), axis=0)

    wih0 = jnp.concatenate([l0_fwd_wih, l0_bwd_wih], axis=1)
    b0 = jnp.concatenate([l0_fwd_b, l0_bwd_b], axis=1)
    ww = wih0[:128].astype(jnp.bfloat16)
    wch = wih0[128:256].astype(jnp.bfloat16)
    mpos = jnp.zeros((64, 8 * _H), jnp.float32)
    mpos = mpos.at[:50].set(pos_emb @ wih0[256:384]).astype(jnp.bfloat16)
    ce_pad = jnp.zeros((128, _CE), jnp.float32).at[:100].set(char_emb)
    mcomb = jnp.concatenate(
        [ce_pad @ char_conv_w[0], ce_pad @ char_conv_w[1],
         ce_pad @ char_conv_w[2]], axis=1).astype(jnp.bfloat16)
    ch = _char_encode(ids, tok_mask, mcomb, char_conv_b)
    gx0 = _proj0(ids, word_x, ch, ww, wch, mpos, b0)
    gx0 = gx0.reshape(_T, _B, 8 * _H)
    mask_tb1 = mask_t.reshape(_T, _B, 1)
    h0f, h0b = _bilstm_layer(gx0, mask_tb1,
                             l0_fwd_whh.astype(jnp.bfloat16),
                             l0_bwd_whh.astype(jnp.bfloat16))

    wih1 = jnp.concatenate([l1_fwd_wih, l1_bwd_wih],
                           axis=1).astype(jnp.bfloat16)
    b1 = jnp.concatenate([l1_fwd_b, l1_bwd_b], axis=1)
    gx1 = _row_linear2(h0f.reshape(n, _H), h0b.reshape(n, _H),
                       wih1[:_H], wih1[_H:], b1).reshape(_T, _B, 8 * _H)
    h1f, h1b = _bilstm_layer(gx1, mask_tb1,
                             l1_fwd_whh.astype(jnp.bfloat16),
                             l1_bwd_whh.astype(jnp.bfloat16))

    bidx = jnp.arange(_B)
    hsse = jnp.concatenate(
        [h1f[subj_start_position, bidx], h1b[subj_start_position, bidx],
         h1f[subj_end_position, bidx], h1b[subj_end_position, bidx]],
        axis=1)
    logits = _fused_heads(h1f, h1b, mask_tb1, hsse, subj_w1, subj_b1, subj_w2,
                          subj_b2, obj_w1, obj_b1, obj_w2, obj_b2)
    logits = jnp.transpose(logits, (1, 0, 2))
    return (logits[:, :, 0], logits[:, :, 1], logits[:, :, 2], logits[:, :, 3])
